# baseline (speedup 1.0000x reference)
import functools

import jax
import jax.numpy as jnp
from jax import lax
from jax.experimental import pallas as pl
from jax.experimental.pallas import tpu as pltpu
from jax.experimental.pallas import tpu_sc as plsc

_D = 1024
_DFF = 4096
_NE = 8
_N = 4096
_NP = 2 * _N
_TB = 512
_PPAD = _NP + _NE * _TB
_NBLK = _PPAD // _TB
_FB = 1024
_NF = _DFF // _FB
_RB = 512
_GW = 128
_NRB = _N // _RB

_NW = 32
_DCH = 64
_DNC = (_NP // _NW) // _DCH
_CCH = 32
_CNC = (_N // _NW) // _CCH


def _router_kernel(x_ref, rw_ref, temp_ref, grep0_ref, grep1_ref, pos0_ref,
                   pos1_ref, bex_ref, xbf_ref, cnt_ref, e0s, e1s, gr0s,
                   gr1s):
    tb = pl.program_id(0)

    @pl.when(tb == 0)
    def _():
        cnt_ref[...] = jnp.zeros_like(cnt_ref)

    @pl.when(tb < _NRB)
    def _():
        x = x_ref[...]
        xbf_ref[...] = x.astype(jnp.bfloat16)
        rw = rw_ref[...]
        logits = lax.dot_general(
            x, rw, (((1,), (1,)), ((), ())),
            preferred_element_type=jnp.float32)
        logits = logits / temp_ref[0, 0]
        iota = lax.broadcasted_iota(jnp.int32, logits.shape, 1)
        l0 = jnp.max(logits, axis=1, keepdims=True)
        i0 = jnp.min(jnp.where(logits == l0, iota, _NE), axis=1,
                     keepdims=True)
        lm = jnp.where(iota == i0, -jnp.inf, logits)
        l1 = jnp.max(lm, axis=1, keepdims=True)
        i1 = jnp.min(jnp.where(lm == l1, iota, _NE), axis=1, keepdims=True)
        p0 = jax.nn.sigmoid(l0 - l1)
        p1 = jax.nn.sigmoid(l1 - l0)
        grep0_ref[...] = jnp.broadcast_to(p0, (_RB, _GW))
        grep1_ref[...] = jnp.broadcast_to(p1, (_RB, _GW))
        e0s[pl.ds(tb * _RB, _RB)] = i0[:, 0]
        e1s[pl.ds(tb * _RB, _RB)] = i1[:, 0]

        oh0 = (iota == i0).astype(jnp.float32)
        oh1 = (iota == i1).astype(jnp.float32)
        ir = lax.broadcasted_iota(jnp.int32, (_RB, _RB), 0)
        ic = lax.broadcasted_iota(jnp.int32, (_RB, _RB), 1)
        ltm = (ic < ir).astype(jnp.bfloat16)
        cum0 = lax.dot_general(
            ltm, oh0.astype(jnp.bfloat16), (((1,), (0,)), ((), ())),
            preferred_element_type=jnp.float32)
        cum1 = lax.dot_general(
            ltm, oh1.astype(jnp.bfloat16), (((1,), (0,)), ((), ())),
            preferred_element_type=jnp.float32)
        total0 = jnp.sum(oh0, axis=0, keepdims=True)
        total1 = jnp.sum(oh1, axis=0, keepdims=True)
        cnt = cnt_ref[...]
        grank0 = jnp.sum((cum0 + cnt) * oh0, axis=1)
        grank1 = jnp.sum((cum1 + total0 + cnt) * oh1, axis=1)
        gr0s[pl.ds(tb * _RB, _RB)] = grank0.astype(jnp.int32)
        gr1s[pl.ds(tb * _RB, _RB)] = grank1.astype(jnp.int32)
        cnt_ref[...] = cnt + total0 + total1

    @pl.when(tb == _NRB)
    def _():
        cnts = cnt_ref[...]
        padded = jnp.ceil(cnts / _TB) * _TB
        ir8 = lax.broadcasted_iota(jnp.int32, (_NE, _NE), 0)
        ic8 = lax.broadcasted_iota(jnp.int32, (_NE, _NE), 1)
        excl = (ir8 < ic8).astype(jnp.bfloat16)
        offs = lax.dot_general(
            padded.astype(jnp.bfloat16), excl, (((1,), (0,)), ((), ())),
            preferred_element_type=jnp.float32)

        def pick(e_col):
            it = lax.broadcasted_iota(jnp.int32, (_N, _NE), 1)
            oh = (it == e_col[:, None]).astype(jnp.float32)
            return jnp.sum(oh * offs, axis=1).astype(jnp.int32)

        pos0_ref[...] = pick(e0s[...]) + gr0s[...]
        pos1_ref[...] = pick(e1s[...]) + gr1s[...]

        ob = offs / _TB
        bio = lax.broadcasted_iota(jnp.int32, (_NBLK, _NE), 0)
        cmp = (ob <= bio.astype(jnp.float32)).astype(jnp.float32)
        bex = jnp.sum(cmp, axis=1).astype(jnp.int32) - 1
        bex_ref[...] = jnp.clip(bex, 0, _NE - 1)


def _router(xf, router_w, temp):
    last = _NRB - 1
    return pl.pallas_call(
        _router_kernel,
        grid=(_NRB + 1,),
        in_specs=[
            pl.BlockSpec((_RB, _D), lambda tb: (jnp.minimum(tb, last), 0)),
            pl.BlockSpec((_NE, _D), lambda tb: (0, 0)),
            pl.BlockSpec((1, 1), lambda tb: (0, 0)),
        ],
        out_specs=[
            pl.BlockSpec((_RB, _GW), lambda tb: (jnp.minimum(tb, last), 0)),
            pl.BlockSpec((_RB, _GW), lambda tb: (jnp.minimum(tb, last), 0)),
            pl.BlockSpec((_N,), lambda tb: (0,)),
            pl.BlockSpec((_N,), lambda tb: (0,)),
            pl.BlockSpec((_NBLK,), lambda tb: (0,)),
            pl.BlockSpec((_RB, _D), lambda tb: (jnp.minimum(tb, last), 0)),
        ],
        out_shape=[
            jax.ShapeDtypeStruct((_N, _GW), jnp.float32),
            jax.ShapeDtypeStruct((_N, _GW), jnp.float32),
            jax.ShapeDtypeStruct((_N,), jnp.int32),
            jax.ShapeDtypeStruct((_N,), jnp.int32),
            jax.ShapeDtypeStruct((_NBLK,), jnp.int32),
            jax.ShapeDtypeStruct((_N, _D), jnp.bfloat16),
        ],
        scratch_shapes=[
            pltpu.VMEM((1, _NE), jnp.float32),
            pltpu.VMEM((_N,), jnp.int32),
            pltpu.VMEM((_N,), jnp.int32),
            pltpu.VMEM((_N,), jnp.int32),
            pltpu.VMEM((_N,), jnp.int32),
        ],
    )(xf, router_w, temp)


def _dispatch_sc(xf, grep0, grep1, p0r, p1r):
    mesh = plsc.VectorSubcoreMesh(core_axis_name="c", subcore_axis_name="s")

    @functools.partial(
        pl.kernel, mesh=mesh,
        out_type=[
            jax.ShapeDtypeStruct((_PPAD, _D // 2), jnp.int32),
            jax.ShapeDtypeStruct((_PPAD, _GW), jnp.float32),
        ],
        scratch_types=[
            pltpu.VMEM((_DCH, _D // 2), jnp.int32),
            pltpu.VMEM((_DCH, _GW), jnp.float32),
            pltpu.VMEM((_DNC, _DCH), jnp.int32),
            pltpu.SemaphoreType.DMA,
            pltpu.SemaphoreType.DMA,
        ],
    )
    def k(xf_hbm, g0_hbm, g1_hbm, p0_hbm, p1_hbm, xs_hbm, gs_hbm, rows_v,
          gbuf_v, idx_v, sem0, sem1):
        wid = lax.axis_index("s") * 2 + lax.axis_index("c")
        widk = lax.rem(wid, 16)
        tok_base = widk * (_DNC * _DCH)

        def go(g_hbm, p_hbm):
            pltpu.sync_copy(p_hbm.at[widk], idx_v)
            for c in range(_DNC):
                pltpu.sync_copy(xf_hbm.at[pl.ds(tok_base + c * _DCH, _DCH)],
                                rows_v)
                pltpu.sync_copy(g_hbm.at[pl.ds(tok_base + c * _DCH, _DCH)],
                                gbuf_v)
                cp0 = pltpu.async_copy(rows_v, xs_hbm.at[idx_v.at[c]], sem0)
                cp1 = pltpu.async_copy(gbuf_v, gs_hbm.at[idx_v.at[c]], sem1)
                cp0.wait()
                cp1.wait()

        @pl.when(wid < 16)
        def _():
            go(g0_hbm, p0_hbm)

        @pl.when(wid >= 16)
        def _():
            go(g1_hbm, p1_hbm)

    return k(xf, grep0, grep1, p0r, p1r)


def _combine_sc(eo, p03, p13):
    mesh = plsc.VectorSubcoreMesh(core_axis_name="c", subcore_axis_name="s")

    @functools.partial(
        pl.kernel, mesh=mesh,
        out_type=jax.ShapeDtypeStruct((_N, _D), jnp.float32),
        scratch_types=[
            pltpu.VMEM((_CCH, _D), jnp.float32),
            pltpu.VMEM((_CCH, _D), jnp.float32),
            pltpu.VMEM((_CNC, _CCH), jnp.int32),
            pltpu.VMEM((_CNC, _CCH), jnp.int32),
            pltpu.SemaphoreType.DMA,
            pltpu.SemaphoreType.DMA,
        ],
    )
    def k(eo_hbm, p0_hbm, p1_hbm, out_hbm, buf0, buf1, idx0, idx1, sem0,
          sem1):
        wid = lax.axis_index("s") * 2 + lax.axis_index("c")
        pltpu.sync_copy(p0_hbm.at[wid], idx0)
        pltpu.sync_copy(p1_hbm.at[wid], idx1)
        tok_base = wid * (_CNC * _CCH)
        for c in range(_CNC):
            cp0 = pltpu.async_copy(eo_hbm.at[idx0.at[c]], buf0, sem0)
            cp1 = pltpu.async_copy(eo_hbm.at[idx1.at[c]], buf1, sem1)
            cp0.wait()
            cp1.wait()

            def body(i, _):
                for j in range(0, _D, 16):
                    buf0[i, pl.ds(j, 16)] = (buf0[i, pl.ds(j, 16)]
                                             + buf1[i, pl.ds(j, 16)])
                return 0

            lax.fori_loop(0, _CCH, body, 0)
            pltpu.sync_copy(buf0,
                            out_hbm.at[pl.ds(tok_base + c * _CCH, _CCH)])

    return k(eo, p03, p13)


def _ffn_kernel(bex_ref, xs_ref, gs_ref, w1a_ref, w1g_ref, b1_ref, w2_ref,
                b2_ref, out_ref):
    b = pl.program_id(0)
    f = pl.program_id(1)
    e = bex_ref[b]
    xb = xs_ref[...]
    w1a = w1a_ref[0].astype(jnp.bfloat16)
    w1g = w1g_ref[0].astype(jnp.bfloat16)
    b1a = b1_ref[pl.ds(e, 1), pl.ds(f * _FB, _FB)]
    b1g = b1_ref[pl.ds(e, 1), pl.ds(_DFF + f * _FB, _FB)]
    a = lax.dot_general(
        xb, w1a, (((1,), (1,)), ((), ())), preferred_element_type=jnp.float32)
    a = a + b1a
    g = lax.dot_general(
        xb, w1g, (((1,), (1,)), ((), ())), preferred_element_type=jnp.float32)
    g = g + b1g
    h2 = a * (g * jax.nn.sigmoid(g))
    w2b = w2_ref[0].astype(jnp.bfloat16)
    part = lax.dot_general(
        h2.astype(jnp.bfloat16), w2b, (((1,), (1,)), ((), ())),
        preferred_element_type=jnp.float32)
    gate = gs_ref[:, 0]
    contrib = part * gate[:, None]
    b2row = b2_ref[pl.ds(e, 1), :]
    contrib = contrib + jnp.where(f == 0, 1.0, 0.0) * (gate[:, None] * b2row)

    @pl.when(f == 0)
    def _():
        out_ref[...] = contrib

    @pl.when(f != 0)
    def _():
        out_ref[...] = out_ref[...] + contrib


def _ffn(bex, xs, gs, w1, b1, w2, b2):
    grid_spec = pltpu.PrefetchScalarGridSpec(
        num_scalar_prefetch=1,
        grid=(_NBLK, _NF),
        in_specs=[
            pl.BlockSpec((_TB, _D), lambda b, f, bex: (b, 0)),
            pl.BlockSpec((_TB, _GW), lambda b, f, bex: (b, 0)),
            pl.BlockSpec((1, _FB, _D), lambda b, f, bex: (bex[b], f, 0)),
            pl.BlockSpec((1, _FB, _D), lambda b, f, bex: (bex[b], _NF + f, 0)),
            pl.BlockSpec((_NE, 2 * _DFF), lambda b, f, bex: (0, 0)),
            pl.BlockSpec((1, _D, _FB), lambda b, f, bex: (bex[b], 0, f)),
            pl.BlockSpec((_NE, _D), lambda b, f, bex: (0, 0)),
        ],
        out_specs=pl.BlockSpec((_TB, _D), lambda b, f, bex: (b, 0)),
    )
    return pl.pallas_call(
        _ffn_kernel,
        grid_spec=grid_spec,
        out_shape=jax.ShapeDtypeStruct((_PPAD, _D), jnp.float32),
    )(bex, xs, gs, w1, w1, b1, w2, b2)


def kernel(x, router_w, router_temp, w1, b1, w2, b2):
    B, S, D = x.shape
    xf = x.reshape(-1, D)
    temp = router_temp.reshape(1, 1)
    grep0, grep1, pos0, pos1, bex, xbf = _router(xf, router_w, temp)
    xi = lax.bitcast_convert_type(
        xbf.reshape(_N, _D // 2, 2), jnp.int32)
    xsi, gs = _dispatch_sc(xi, grep0, grep1,
                           pos0.reshape(16, _DNC, _DCH),
                           pos1.reshape(16, _DNC, _DCH))
    xs = lax.bitcast_convert_type(xsi, jnp.bfloat16).reshape(_PPAD, _D)
    eo = _ffn(bex, xs, gs, w1, b1, w2, b2)
    p03 = pos0.reshape(_NW, _CNC, _CCH)
    p13 = pos1.reshape(_NW, _CNC, _CCH)
    out = _combine_sc(eo, p03, p13)
    return out.reshape(B, S, D)


# speedup vs baseline: 1.5763x; 1.5763x over previous
import functools

import jax
import jax.numpy as jnp
from jax import lax
from jax.experimental import pallas as pl
from jax.experimental.pallas import tpu as pltpu
from jax.experimental.pallas import tpu_sc as plsc

_D = 1024
_DFF = 4096
_NE = 8
_N = 4096
_NP = 2 * _N
_TB = 512
_PPAD = _NP + _NE * _TB
_NBLK = _PPAD // _TB
_FB = 1024
_NF = _DFF // _FB
_RB = 512
_GW = 128
_NRB = _N // _RB

_NW = 32
_DCH = 64
_DNC = (_NP // _NW) // _DCH
_CCH = 32
_CNC = (_N // _NW) // _CCH


def _router_kernel(x_ref, rw_ref, temp_ref, grep0_ref, grep1_ref, pos0_ref,
                   pos1_ref, bex_ref, cnt_ref, e0s, e1s, gr0s, gr1s):
    tb = pl.program_id(0)

    @pl.when(tb == 0)
    def _():
        cnt_ref[...] = jnp.zeros_like(cnt_ref)

    @pl.when(tb < _NRB)
    def _():
        x = x_ref[...]
        rw = rw_ref[...]
        logits = lax.dot_general(
            x, rw, (((1,), (1,)), ((), ())),
            preferred_element_type=jnp.float32)
        logits = logits / temp_ref[0, 0]
        iota = lax.broadcasted_iota(jnp.int32, logits.shape, 1)
        l0 = jnp.max(logits, axis=1, keepdims=True)
        i0 = jnp.min(jnp.where(logits == l0, iota, _NE), axis=1,
                     keepdims=True)
        lm = jnp.where(iota == i0, -jnp.inf, logits)
        l1 = jnp.max(lm, axis=1, keepdims=True)
        i1 = jnp.min(jnp.where(lm == l1, iota, _NE), axis=1, keepdims=True)
        p0 = jax.nn.sigmoid(l0 - l1)
        p1 = jax.nn.sigmoid(l1 - l0)
        grep0_ref[...] = jnp.broadcast_to(p0, (_RB, _GW))
        grep1_ref[...] = jnp.broadcast_to(p1, (_RB, _GW))
        e0s[pl.ds(tb * _RB, _RB)] = i0[:, 0]
        e1s[pl.ds(tb * _RB, _RB)] = i1[:, 0]

        oh0 = (iota == i0).astype(jnp.float32)
        oh1 = (iota == i1).astype(jnp.float32)
        ir = lax.broadcasted_iota(jnp.int32, (_RB, _RB), 0)
        ic = lax.broadcasted_iota(jnp.int32, (_RB, _RB), 1)
        ltm = (ic < ir).astype(jnp.bfloat16)
        cum0 = lax.dot_general(
            ltm, oh0.astype(jnp.bfloat16), (((1,), (0,)), ((), ())),
            preferred_element_type=jnp.float32)
        cum1 = lax.dot_general(
            ltm, oh1.astype(jnp.bfloat16), (((1,), (0,)), ((), ())),
            preferred_element_type=jnp.float32)
        total0 = jnp.sum(oh0, axis=0, keepdims=True)
        total1 = jnp.sum(oh1, axis=0, keepdims=True)
        cnt = cnt_ref[...]
        grank0 = jnp.sum((cum0 + cnt) * oh0, axis=1)
        grank1 = jnp.sum((cum1 + total0 + cnt) * oh1, axis=1)
        gr0s[pl.ds(tb * _RB, _RB)] = grank0.astype(jnp.int32)
        gr1s[pl.ds(tb * _RB, _RB)] = grank1.astype(jnp.int32)
        cnt_ref[...] = cnt + total0 + total1

    @pl.when(tb == _NRB)
    def _():
        cnts = cnt_ref[...]
        padded = jnp.ceil(cnts / _TB) * _TB
        ir8 = lax.broadcasted_iota(jnp.int32, (_NE, _NE), 0)
        ic8 = lax.broadcasted_iota(jnp.int32, (_NE, _NE), 1)
        excl = (ir8 < ic8).astype(jnp.bfloat16)
        offs = lax.dot_general(
            padded.astype(jnp.bfloat16), excl, (((1,), (0,)), ((), ())),
            preferred_element_type=jnp.float32)

        def pick(e_col):
            it = lax.broadcasted_iota(jnp.int32, (_N, _NE), 1)
            oh = (it == e_col[:, None]).astype(jnp.float32)
            return jnp.sum(oh * offs, axis=1).astype(jnp.int32)

        pos0_ref[...] = pick(e0s[...]) + gr0s[...]
        pos1_ref[...] = pick(e1s[...]) + gr1s[...]

        ob = offs / _TB
        bio = lax.broadcasted_iota(jnp.int32, (_NBLK, _NE), 0)
        cmp = (ob <= bio.astype(jnp.float32)).astype(jnp.float32)
        bex = jnp.sum(cmp, axis=1).astype(jnp.int32) - 1
        bex_ref[...] = jnp.clip(bex, 0, _NE - 1)


def _router(xf, router_w, temp):
    last = _NRB - 1
    return pl.pallas_call(
        _router_kernel,
        grid=(_NRB + 1,),
        in_specs=[
            pl.BlockSpec((_RB, _D), lambda tb: (jnp.minimum(tb, last), 0)),
            pl.BlockSpec((_NE, _D), lambda tb: (0, 0)),
            pl.BlockSpec((1, 1), lambda tb: (0, 0)),
        ],
        out_specs=[
            pl.BlockSpec((_RB, _GW), lambda tb: (jnp.minimum(tb, last), 0)),
            pl.BlockSpec((_RB, _GW), lambda tb: (jnp.minimum(tb, last), 0)),
            pl.BlockSpec((_N,), lambda tb: (0,)),
            pl.BlockSpec((_N,), lambda tb: (0,)),
            pl.BlockSpec((_NBLK,), lambda tb: (0,)),
        ],
        out_shape=[
            jax.ShapeDtypeStruct((_N, _GW), jnp.float32),
            jax.ShapeDtypeStruct((_N, _GW), jnp.float32),
            jax.ShapeDtypeStruct((_N,), jnp.int32),
            jax.ShapeDtypeStruct((_N,), jnp.int32),
            jax.ShapeDtypeStruct((_NBLK,), jnp.int32),
        ],
        scratch_shapes=[
            pltpu.VMEM((1, _NE), jnp.float32),
            pltpu.VMEM((_N,), jnp.int32),
            pltpu.VMEM((_N,), jnp.int32),
            pltpu.VMEM((_N,), jnp.int32),
            pltpu.VMEM((_N,), jnp.int32),
        ],
    )(xf, router_w, temp)


def _dispatch_sc(xf, grep0, grep1, p0r, p1r):
    mesh = plsc.VectorSubcoreMesh(core_axis_name="c", subcore_axis_name="s")

    @functools.partial(
        pl.kernel, mesh=mesh,
        out_type=[
            jax.ShapeDtypeStruct((_PPAD, _D), jnp.float32),
            jax.ShapeDtypeStruct((_PPAD, _GW), jnp.float32),
        ],
        scratch_types=[
            pltpu.VMEM((_DCH, _D), jnp.float32),
            pltpu.VMEM((_DCH, _GW), jnp.float32),
            pltpu.VMEM((_DNC, _DCH), jnp.int32),
            pltpu.SemaphoreType.DMA,
            pltpu.SemaphoreType.DMA,
        ],
    )
    def k(xf_hbm, g0_hbm, g1_hbm, p0_hbm, p1_hbm, xs_hbm, gs_hbm, rows_v,
          gbuf_v, idx_v, sem0, sem1):
        wid = lax.axis_index("s") * 2 + lax.axis_index("c")
        widk = lax.rem(wid, 16)
        tok_base = widk * (_DNC * _DCH)

        def go(g_hbm, p_hbm):
            pltpu.sync_copy(p_hbm.at[widk], idx_v)
            for c in range(_DNC):
                pltpu.sync_copy(xf_hbm.at[pl.ds(tok_base + c * _DCH, _DCH)],
                                rows_v)
                pltpu.sync_copy(g_hbm.at[pl.ds(tok_base + c * _DCH, _DCH)],
                                gbuf_v)
                cp0 = pltpu.async_copy(rows_v, xs_hbm.at[idx_v.at[c]], sem0)
                cp1 = pltpu.async_copy(gbuf_v, gs_hbm.at[idx_v.at[c]], sem1)
                cp0.wait()
                cp1.wait()

        @pl.when(wid < 16)
        def _():
            go(g0_hbm, p0_hbm)

        @pl.when(wid >= 16)
        def _():
            go(g1_hbm, p1_hbm)

    return k(xf, grep0, grep1, p0r, p1r)


def _combine_sc(eo, p03, p13):
    mesh = plsc.VectorSubcoreMesh(core_axis_name="c", subcore_axis_name="s")

    @functools.partial(
        pl.kernel, mesh=mesh,
        out_type=jax.ShapeDtypeStruct((_N, _D), jnp.float32),
        scratch_types=[
            pltpu.VMEM((_CCH, _D), jnp.float32),
            pltpu.VMEM((_CCH, _D), jnp.float32),
            pltpu.VMEM((_CNC, _CCH), jnp.int32),
            pltpu.VMEM((_CNC, _CCH), jnp.int32),
            pltpu.SemaphoreType.DMA,
            pltpu.SemaphoreType.DMA,
        ],
    )
    def k(eo_hbm, p0_hbm, p1_hbm, out_hbm, buf0, buf1, idx0, idx1, sem0,
          sem1):
        wid = lax.axis_index("s") * 2 + lax.axis_index("c")
        pltpu.sync_copy(p0_hbm.at[wid], idx0)
        pltpu.sync_copy(p1_hbm.at[wid], idx1)
        tok_base = wid * (_CNC * _CCH)
        for c in range(_CNC):
            cp0 = pltpu.async_copy(eo_hbm.at[idx0.at[c]], buf0, sem0)
            cp1 = pltpu.async_copy(eo_hbm.at[idx1.at[c]], buf1, sem1)
            cp0.wait()
            cp1.wait()

            def body(i, _):
                for j in range(0, _D, 16):
                    buf0[i, pl.ds(j, 16)] = (buf0[i, pl.ds(j, 16)]
                                             + buf1[i, pl.ds(j, 16)])
                return 0

            lax.fori_loop(0, _CCH, body, 0)
            pltpu.sync_copy(buf0,
                            out_hbm.at[pl.ds(tok_base + c * _CCH, _CCH)])

    return k(eo, p03, p13)


def _ffn_kernel(bex_ref, xs_ref, gs_ref, w1a_ref, w1g_ref, b1_ref, w2_ref,
                b2_ref, out_ref):
    b = pl.program_id(0)
    f = pl.program_id(1)
    e = bex_ref[b]
    xb = xs_ref[...].astype(jnp.bfloat16)
    w1a = w1a_ref[0].astype(jnp.bfloat16)
    w1g = w1g_ref[0].astype(jnp.bfloat16)
    b1a = b1_ref[pl.ds(e, 1), pl.ds(f * _FB, _FB)]
    b1g = b1_ref[pl.ds(e, 1), pl.ds(_DFF + f * _FB, _FB)]
    a = lax.dot_general(
        xb, w1a, (((1,), (1,)), ((), ())), preferred_element_type=jnp.float32)
    a = a + b1a
    g = lax.dot_general(
        xb, w1g, (((1,), (1,)), ((), ())), preferred_element_type=jnp.float32)
    g = g + b1g
    h2 = a * (g * jax.nn.sigmoid(g))
    w2b = w2_ref[0].astype(jnp.bfloat16)
    part = lax.dot_general(
        h2.astype(jnp.bfloat16), w2b, (((1,), (1,)), ((), ())),
        preferred_element_type=jnp.float32)
    gate = gs_ref[:, 0]
    contrib = part * gate[:, None]
    b2row = b2_ref[pl.ds(e, 1), :]
    contrib = contrib + jnp.where(f == 0, 1.0, 0.0) * (gate[:, None] * b2row)

    @pl.when(f == 0)
    def _():
        out_ref[...] = contrib

    @pl.when(f != 0)
    def _():
        out_ref[...] = out_ref[...] + contrib


def _ffn(bex, xs, gs, w1, b1, w2, b2):
    grid_spec = pltpu.PrefetchScalarGridSpec(
        num_scalar_prefetch=1,
        grid=(_NBLK, _NF),
        in_specs=[
            pl.BlockSpec((_TB, _D), lambda b, f, bex: (b, 0)),
            pl.BlockSpec((_TB, _GW), lambda b, f, bex: (b, 0)),
            pl.BlockSpec((1, _FB, _D), lambda b, f, bex: (bex[b], f, 0)),
            pl.BlockSpec((1, _FB, _D), lambda b, f, bex: (bex[b], _NF + f, 0)),
            pl.BlockSpec((_NE, 2 * _DFF), lambda b, f, bex: (0, 0)),
            pl.BlockSpec((1, _D, _FB), lambda b, f, bex: (bex[b], 0, f)),
            pl.BlockSpec((_NE, _D), lambda b, f, bex: (0, 0)),
        ],
        out_specs=pl.BlockSpec((_TB, _D), lambda b, f, bex: (b, 0)),
    )
    return pl.pallas_call(
        _ffn_kernel,
        grid_spec=grid_spec,
        out_shape=jax.ShapeDtypeStruct((_PPAD, _D), jnp.float32),
    )(bex, xs, gs, w1, w1, b1, w2, b2)


def kernel(x, router_w, router_temp, w1, b1, w2, b2):
    B, S, D = x.shape
    xf = x.reshape(-1, D)
    temp = router_temp.reshape(1, 1)
    grep0, grep1, pos0, pos1, bex = _router(xf, router_w, temp)
    xs, gs = _dispatch_sc(xf, grep0, grep1,
                          pos0.reshape(16, _DNC, _DCH),
                          pos1.reshape(16, _DNC, _DCH))
    eo = _ffn(bex, xs, gs, w1, b1, w2, b2)
    p03 = pos0.reshape(_NW, _CNC, _CCH)
    p13 = pos1.reshape(_NW, _CNC, _CCH)
    out = _combine_sc(eo, p03, p13)
    return out.reshape(B, S, D)
